# Initial kernel scaffold; baseline (speedup 1.0000x reference)
import jax
import jax.numpy as jnp
from jax.experimental import pallas as pl


def kernel(x, edge_index, W_l, b_l, W_r, b_r, att, bias, W1, b1, W2, b2):
    raise NotImplementedError("write your pallas kernel here")


# speedup vs baseline: 8.9188x; 8.9188x over previous
import functools

import jax
import jax.numpy as jnp
from jax import lax
from jax.experimental import pallas as pl
from jax.experimental.pallas import tpu as pltpu
from jax.experimental.pallas import tpu_sc as plsc

_N = 10000
_E = 320000
_H = 8
_C = 32
_HC = _H * _C
_NSUB = 16
_NCORE = 2
_B1 = 80
_CH1 = (_E // 32) // _B1
_B2 = 80
_CH2 = (_E // _NSUB) // _B2
_NP = 10240
_RPS = _NP // _NSUB


def _proj_body(x_ref, wl_ref, wr_ref, bl_ref, br_ref, xl_ref, xr_ref):
    x = x_ref[...]
    xl_ref[...] = jnp.dot(x, wl_ref[...],
                          preferred_element_type=jnp.float32) + bl_ref[...]
    xr_ref[...] = jnp.dot(x, wr_ref[...],
                          preferred_element_type=jnp.float32) + br_ref[...]


def _proj(x, W_l, W_r, bl2, br2):
    blk = 400
    grid = (_N // blk,)
    return pl.pallas_call(
        _proj_body,
        grid=grid,
        in_specs=[
            pl.BlockSpec((blk, 128), lambda i: (i, 0)),
            pl.BlockSpec((128, _HC), lambda i: (0, 0)),
            pl.BlockSpec((128, _HC), lambda i: (0, 0)),
            pl.BlockSpec((1, _HC), lambda i: (0, 0)),
            pl.BlockSpec((1, _HC), lambda i: (0, 0)),
        ],
        out_specs=[
            pl.BlockSpec((blk, _HC), lambda i: (i, 0)),
            pl.BlockSpec((blk, _HC), lambda i: (i, 0)),
        ],
        out_shape=[
            jax.ShapeDtypeStruct((_N, _HC), jnp.float32),
            jax.ShapeDtypeStruct((_N, _HC), jnp.float32),
        ],
    )(x, W_l, W_r, bl2, br2)


def _pass1_body(xl_hbm, xr_hbm, src_hbm, dst_hbm, att_hbm,
                s_hbm, dp_hbm,
                srcb, dstb, bufL, bufR, sbuf, s16, attb, stg,
                den_sh, sem0, sem1):
    c = lax.axis_index("c")
    sidx = lax.axis_index("s")
    wid = sidx * _NCORE + c
    i32 = jnp.int32
    zero16 = jnp.zeros((16,), jnp.float32)
    iota = lax.iota(i32, 16)

    pltpu.sync_copy(att_hbm, attb)

    def _z_stg(i, _):
        stg[i, :] = zero16
        return 0
    lax.fori_loop(0, _RPS, _z_stg, 0)

    def _z_s16(i, _):
        s16[i, :] = zero16
        return 0
    lax.fori_loop(0, _B1, _z_s16, 0)

    pltpu.sync_copy(stg, den_sh.at[pl.ds(sidx * _RPS, _RPS)])
    plsc.subcore_barrier()

    def _chunk(k, _):
        base = wid * (_E // 32) + k * _B1
        pltpu.sync_copy(src_hbm.at[pl.ds(base, _B1)], srcb.at[0])
        pltpu.sync_copy(dst_hbm.at[pl.ds(base, _B1)], dstb.at[0])
        cl = pltpu.async_copy(xl_hbm.at[srcb.at[0]], bufL, sem0)
        cr = pltpu.async_copy(xr_hbm.at[dstb.at[0]], bufR, sem1)
        cl.wait()
        cr.wait()

        def _group(g, _g):
            rows = g * 16 + iota

            def _head(h, _h):
                def _colq(q, acc):
                    for u in range(4):
                        j = h * _C + q * 4 + u
                        cols = jnp.broadcast_to(j, (16,)).astype(i32)
                        vl = plsc.load_gather(bufL, [rows, cols])
                        vr = plsc.load_gather(bufR, [rows, cols])
                        e = vl + vr
                        e = jnp.maximum(e, 0.2 * e)
                        av = plsc.load_gather(attb, [cols])
                        acc = acc + e * av
                    return acc

                acc = lax.fori_loop(0, _C // 4, _colq, zero16)
                s = jnp.exp(acc)
                hcol = jnp.broadcast_to(h, (16,)).astype(i32)
                plsc.store_scatter(sbuf, [rows, hcol], s)
                plsc.store_scatter(s16, [rows, hcol], s)
                return 0

            lax.fori_loop(0, _H, _head, 0)
            return 0

        lax.fori_loop(0, _B1 // 16, _group, 0)
        pltpu.sync_copy(sbuf, s_hbm.at[pl.ds(base, _B1)])
        pltpu.sync_copy(s16, den_sh.at[dstb.at[0]], add=True)
        return 0

    lax.fori_loop(0, _CH1, _chunk, 0)
    plsc.subcore_barrier()
    r0 = sidx * _RPS
    pltpu.sync_copy(den_sh.at[pl.ds(r0, _RPS)], stg)
    pltpu.sync_copy(stg, dp_hbm.at[c, pl.ds(r0, _RPS)])


@functools.partial(
    pl.kernel,
    out_type=[
        jax.ShapeDtypeStruct((_E, _H), jnp.float32),
        jax.ShapeDtypeStruct((_NCORE, _NP, 16), jnp.float32),
    ],
    mesh=plsc.VectorSubcoreMesh(core_axis_name="c", subcore_axis_name="s"),
    compiler_params=pltpu.CompilerParams(use_tc_tiling_on_sc=False, needs_layout_passes=False),
    scratch_types=[
        pltpu.VMEM((1, _B1), jnp.int32),
        pltpu.VMEM((1, _B1), jnp.int32),
        pltpu.VMEM((_B1, _HC), jnp.float32),
        pltpu.VMEM((_B1, _HC), jnp.float32),
        pltpu.VMEM((_B1, _H), jnp.float32),
        pltpu.VMEM((_B1, 16), jnp.float32),
        pltpu.VMEM((_HC,), jnp.float32),
        pltpu.VMEM((_RPS, 16), jnp.float32),
        pltpu.VMEM_SHARED((_NP, 16), jnp.float32),
        pltpu.SemaphoreType.DMA,
        pltpu.SemaphoreType.DMA,
    ],
)
def _pass1(xl_hbm, xr_hbm, src_hbm, dst_hbm, att_hbm, s_hbm, dp_hbm,
           srcb, dstb, bufL, bufR, sbuf, s16, attb, stg,
           den_sh, sem0, sem1):
    _pass1_body(xl_hbm, xr_hbm, src_hbm, dst_hbm, att_hbm, s_hbm, dp_hbm,
                srcb, dstb, bufL, bufR, sbuf, s16, attb, stg,
                den_sh, sem0, sem1)


def _comb_body(dp_ref, out_ref):
    out_ref[...] = dp_ref[0] + dp_ref[1]


def _combine(dp3):
    return pl.pallas_call(
        _comb_body,
        out_shape=jax.ShapeDtypeStruct((_NP * 16 // 128, 128), jnp.float32),
    )(dp3)


def _pass2_body(xl2_hbm, s_hbm, den_hbm, src_hbm, dst_hbm, out_hbm,
                srcb, dstb, xib, xbuf, msg, dbuf, sbuf, abuf, stg,
                acc_sh, sem0, sem1):
    c = lax.axis_index("c")
    sidx = lax.axis_index("s")
    i32 = jnp.int32
    zero16 = jnp.zeros((16,), jnp.float32)
    iota = lax.iota(i32, 16)

    def _z_stg(i, _):
        for u in range(8):
            stg[i, pl.ds(u * 16, 16)] = zero16
        return 0
    lax.fori_loop(0, _RPS // 5, _z_stg, 0)
    for t in range(5):
        pltpu.sync_copy(
            stg, acc_sh.at[pl.ds(sidx * _RPS + t * (_RPS // 5), _RPS // 5)])
    plsc.subcore_barrier()

    def _chunk(k, _):
        base = sidx * (_E // _NSUB) + k * _B2
        pltpu.sync_copy(src_hbm.at[pl.ds(base, _B2)], srcb.at[0])
        pltpu.sync_copy(dst_hbm.at[pl.ds(base, _B2)], dstb.at[0])

        def _mkidx(g, _g):
            v = srcb[0, pl.ds(g * 16, 16)]
            xib[0, pl.ds(g * 16, 16)] = v * 2 + c
            return 0
        lax.fori_loop(0, _B2 // 16, _mkidx, 0)

        cx = pltpu.async_copy(xl2_hbm.at[xib.at[0]], xbuf, sem0)
        cd = pltpu.async_copy(den_hbm.at[dstb.at[0]], dbuf, sem1)
        pltpu.sync_copy(s_hbm.at[pl.ds(base, _B2)], sbuf)
        cx.wait()
        cd.wait()

        def _alpha(g, _g):
            rows = g * 16 + iota
            for hh in range(4):
                h = (4 * c + hh).astype(i32)
                hv = jnp.broadcast_to(h, (16,))
                sv = plsc.load_gather(sbuf, [rows, hv])
                dv = plsc.load_gather(dbuf, [rows, hv])
                a = sv / (dv + 1e-16)
                hhv = jnp.broadcast_to(hh, (16,)).astype(i32)
                plsc.store_scatter(abuf, [rows, hhv], a)
            return 0
        lax.fori_loop(0, _B2 // 16, _alpha, 0)

        def _edge(e, _e):
            ev = jnp.broadcast_to(e, (16,)).astype(i32)
            for jj in range(8):
                hh = jj // 2
                hhv = jnp.broadcast_to(hh, (16,)).astype(i32)
                a = plsc.load_gather(abuf, [ev, hhv])
                msg[e, pl.ds(jj * 16, 16)] = xbuf[e, pl.ds(jj * 16, 16)] * a
            return 0
        lax.fori_loop(0, _B2, _edge, 0)

        pltpu.sync_copy(msg, acc_sh.at[dstb.at[0]], add=True)
        return 0

    lax.fori_loop(0, _CH2, _chunk, 0)
    plsc.subcore_barrier()
    for t in range(5):
        r0 = sidx * _RPS + t * (_RPS // 5)
        pltpu.sync_copy(acc_sh.at[pl.ds(r0, _RPS // 5)], stg)
        pltpu.sync_copy(stg, out_hbm.at[c, pl.ds(r0, _RPS // 5)])


@functools.partial(
    pl.kernel,
    out_type=jax.ShapeDtypeStruct((_NCORE, _NP, 128), jnp.float32),
    mesh=plsc.VectorSubcoreMesh(core_axis_name="c", subcore_axis_name="s"),
    compiler_params=pltpu.CompilerParams(use_tc_tiling_on_sc=False, needs_layout_passes=False),
    scratch_types=[
        pltpu.VMEM((1, _B2), jnp.int32),
        pltpu.VMEM((1, _B2), jnp.int32),
        pltpu.VMEM((1, _B2), jnp.int32),
        pltpu.VMEM((_B2, 128), jnp.float32),
        pltpu.VMEM((_B2, 128), jnp.float32),
        pltpu.VMEM((_B2, 16), jnp.float32),
        pltpu.VMEM((_B2, _H), jnp.float32),
        pltpu.VMEM((_B2, 4), jnp.float32),
        pltpu.VMEM((_RPS // 5, 128), jnp.float32),
        pltpu.VMEM_SHARED((_NP, 128), jnp.float32),
        pltpu.SemaphoreType.DMA,
        pltpu.SemaphoreType.DMA,
    ],
)
def _pass2(xl2_hbm, s_hbm, den_hbm, src_hbm, dst_hbm, out_hbm,
           srcb, dstb, xib, xbuf, msg, dbuf, sbuf, abuf, stg,
           acc_sh, sem0, sem1):
    _pass2_body(xl2_hbm, s_hbm, den_hbm, src_hbm, dst_hbm, out_hbm,
                srcb, dstb, xib, xbuf, msg, dbuf, sbuf, abuf, stg,
                acc_sh, sem0, sem1)


def _elu(x):
    return jnp.where(x > 0, x, jnp.exp(x) - 1.0)


def _mlp_body(o0_ref, o1_ref, ba_ref, bb_ref, w1a_ref, w1b_ref, b1_ref,
              w2_ref, b2_ref, y_ref):
    h0 = _elu(o0_ref[...] + ba_ref[...])
    h1 = _elu(o1_ref[...] + bb_ref[...])
    t = (jnp.dot(h0, w1a_ref[...], preferred_element_type=jnp.float32)
         + jnp.dot(h1, w1b_ref[...], preferred_element_type=jnp.float32)
         + b1_ref[...])
    t = _elu(t)
    y_ref[...] = jnp.dot(t, w2_ref[...],
                         preferred_element_type=jnp.float32) + b2_ref[...]


def _mlp(o0, o1, ba, bb, w1a, w1b, b12, W2, b22):
    blk = 400
    return pl.pallas_call(
        _mlp_body,
        grid=(_N // blk,),
        in_specs=[
            pl.BlockSpec((blk, 128), lambda i: (i, 0)),
            pl.BlockSpec((blk, 128), lambda i: (i, 0)),
            pl.BlockSpec((1, 128), lambda i: (0, 0)),
            pl.BlockSpec((1, 128), lambda i: (0, 0)),
            pl.BlockSpec((128, _C), lambda i: (0, 0)),
            pl.BlockSpec((128, _C), lambda i: (0, 0)),
            pl.BlockSpec((1, _C), lambda i: (0, 0)),
            pl.BlockSpec((_C, 128), lambda i: (0, 0)),
            pl.BlockSpec((1, 128), lambda i: (0, 0)),
        ],
        out_specs=pl.BlockSpec((blk, 128), lambda i: (i, 0)),
        out_shape=jax.ShapeDtypeStruct((_N, 128), jnp.float32),
    )(o0, o1, ba, bb, w1a, w1b, b12, W2, b22)


def kernel(x, edge_index, W_l, b_l, W_r, b_r, att, bias, W1, b1, W2, b2):
    src = edge_index[0]
    dst = edge_index[1]
    xl, xr = _proj(x, W_l, W_r, b_l.reshape(1, -1), b_r.reshape(1, -1))
    s, dp = _pass1(xl, xr, src, dst, att.reshape(-1))
    den = _combine(dp.reshape(2, _NP * 16 // 128, 128)).reshape(_NP, 16)
    o = _pass2(xl.reshape(2 * _N, 128), s, den, src, dst)
    y = _mlp(o[0, :_N], o[1, :_N],
             bias[:128].reshape(1, 128), bias[128:].reshape(1, 128),
             W1[:128], W1[128:], b1.reshape(1, -1), W2, b2.reshape(1, -1))
    return y


# speedup vs baseline: 16.2829x; 1.8257x over previous
import functools

import jax
import jax.numpy as jnp
from jax import lax
from jax.experimental import pallas as pl
from jax.experimental.pallas import tpu as pltpu
from jax.experimental.pallas import tpu_sc as plsc

_N = 10000
_E = 320000
_H = 8
_C = 32
_HC = _H * _C
_NSUB = 16
_NCORE = 2
_B1 = 80
_CH1 = (_E // 32) // _B1
_B2 = 80
_CH2 = (_E // _NSUB) // _B2
_NP = 10240
_RPS = _NP // _NSUB


def _proj_body(x_ref, wl_ref, wr_ref, bl_ref, br_ref, xl_ref, xr_ref):
    x = x_ref[...]
    xl_ref[...] = jnp.dot(x, wl_ref[...],
                          preferred_element_type=jnp.float32) + bl_ref[...]
    xr_ref[...] = jnp.dot(x, wr_ref[...],
                          preferred_element_type=jnp.float32) + br_ref[...]


def _proj(x, W_l, W_r, bl2, br2):
    blk = 400
    grid = (_N // blk,)
    return pl.pallas_call(
        _proj_body,
        grid=grid,
        in_specs=[
            pl.BlockSpec((blk, 128), lambda i: (i, 0)),
            pl.BlockSpec((128, _HC), lambda i: (0, 0)),
            pl.BlockSpec((128, _HC), lambda i: (0, 0)),
            pl.BlockSpec((1, _HC), lambda i: (0, 0)),
            pl.BlockSpec((1, _HC), lambda i: (0, 0)),
        ],
        out_specs=[
            pl.BlockSpec((blk, _HC), lambda i: (i, 0)),
            pl.BlockSpec((blk, _HC), lambda i: (i, 0)),
        ],
        out_shape=[
            jax.ShapeDtypeStruct((_N, _HC), jnp.float32),
            jax.ShapeDtypeStruct((_N, _HC), jnp.float32),
        ],
    )(x, W_l, W_r, bl2, br2)


def _pass1_body(xl_hbm, xr_hbm, src_hbm, dst_hbm, att_hbm,
                s_hbm, dp_hbm,
                srcb, dstb, bufL, bufR, s16, attb, stg,
                den_sh, sem0, sem1):
    c = lax.axis_index("c")
    sidx = lax.axis_index("s")
    wid = sidx * _NCORE + c
    i32 = jnp.int32
    zero16 = jnp.zeros((16,), jnp.float32)
    iota = lax.iota(i32, 16)

    pltpu.sync_copy(att_hbm, attb)
    attv = [attb[pl.ds(jj * 16, 16)] for jj in range(_HC // 16)]

    def _z_stg(i, _):
        stg[i, :] = zero16
        return 0
    lax.fori_loop(0, _RPS, _z_stg, 0)

    pltpu.sync_copy(stg, den_sh.at[pl.ds(sidx * _RPS, _RPS)])
    plsc.subcore_barrier()

    def _chunk(k, _):
        base = wid * (_E // 32) + k * _B1
        pltpu.sync_copy(src_hbm.at[pl.ds(base, _B1)], srcb.at[0])
        pltpu.sync_copy(dst_hbm.at[pl.ds(base, _B1)], dstb.at[0])
        cl = pltpu.async_copy(xl_hbm.at[srcb.at[0]], bufL, sem0)
        cr = pltpu.async_copy(xr_hbm.at[dstb.at[0]], bufR, sem1)
        cl.wait()
        cr.wait()

        def _edge(e, _e):
            svec = jnp.full((16,), -1e30, jnp.float32)
            for h in range(_H):
                j0 = h * _C
                a0 = bufL[e, pl.ds(j0, 16)] + bufR[e, pl.ds(j0, 16)]
                a1 = bufL[e, pl.ds(j0 + 16, 16)] + bufR[e, pl.ds(j0 + 16, 16)]
                a0 = jnp.maximum(a0, 0.2 * a0) * attv[2 * h]
                a1 = jnp.maximum(a1, 0.2 * a1) * attv[2 * h + 1]
                s = jnp.sum(a0 + a1)
                svec = jnp.where(iota == h, s, svec)
            s16[e, :] = jnp.exp(svec)
            return 0

        lax.fori_loop(0, _B1, _edge, 0)
        pltpu.sync_copy(s16, s_hbm.at[pl.ds(base, _B1)])
        pltpu.sync_copy(s16, den_sh.at[dstb.at[0]], add=True)
        return 0

    lax.fori_loop(0, _CH1, _chunk, 0)
    plsc.subcore_barrier()
    r0 = sidx * _RPS
    pltpu.sync_copy(den_sh.at[pl.ds(r0, _RPS)], stg)
    pltpu.sync_copy(stg, dp_hbm.at[c, pl.ds(r0, _RPS)])


@functools.partial(
    pl.kernel,
    out_type=[
        jax.ShapeDtypeStruct((_E, 16), jnp.float32),
        jax.ShapeDtypeStruct((_NCORE, _NP, 16), jnp.float32),
    ],
    mesh=plsc.VectorSubcoreMesh(core_axis_name="c", subcore_axis_name="s"),
    compiler_params=pltpu.CompilerParams(use_tc_tiling_on_sc=False, needs_layout_passes=False),
    scratch_types=[
        pltpu.VMEM((1, _B1), jnp.int32),
        pltpu.VMEM((1, _B1), jnp.int32),
        pltpu.VMEM((_B1, _HC), jnp.float32),
        pltpu.VMEM((_B1, _HC), jnp.float32),
        pltpu.VMEM((_B1, 16), jnp.float32),
        pltpu.VMEM((_HC,), jnp.float32),
        pltpu.VMEM((_RPS, 16), jnp.float32),
        pltpu.VMEM_SHARED((_NP, 16), jnp.float32),
        pltpu.SemaphoreType.DMA,
        pltpu.SemaphoreType.DMA,
    ],
)
def _pass1(xl_hbm, xr_hbm, src_hbm, dst_hbm, att_hbm, s_hbm, dp_hbm,
           srcb, dstb, bufL, bufR, s16, attb, stg,
           den_sh, sem0, sem1):
    _pass1_body(xl_hbm, xr_hbm, src_hbm, dst_hbm, att_hbm, s_hbm, dp_hbm,
                srcb, dstb, bufL, bufR, s16, attb, stg,
                den_sh, sem0, sem1)


def _comb_body(dp_ref, out_ref):
    out_ref[...] = dp_ref[0] + dp_ref[1]


def _combine(dp3):
    return pl.pallas_call(
        _comb_body,
        out_shape=jax.ShapeDtypeStruct((_NP * 16 // 128, 128), jnp.float32),
    )(dp3)


def _pass2_body(xl2_hbm, s_hbm, den_hbm, src_hbm, dst_hbm, out_hbm,
                srcb, dstb, xib, xbuf, msg, dbuf, sbuf, stg,
                acc_sh, sem0, sem1):
    c = lax.axis_index("c")
    sidx = lax.axis_index("s")
    i32 = jnp.int32
    zero16 = jnp.zeros((16,), jnp.float32)
    iota = lax.iota(i32, 16)

    def _z_stg(i, _):
        for u in range(8):
            stg[i, pl.ds(u * 16, 16)] = zero16
        return 0
    lax.fori_loop(0, _RPS // 5, _z_stg, 0)
    for t in range(5):
        pltpu.sync_copy(
            stg, acc_sh.at[pl.ds(sidx * _RPS + t * (_RPS // 5), _RPS // 5)])
    plsc.subcore_barrier()

    def _chunk(k, _):
        base = sidx * (_E // _NSUB) + k * _B2
        pltpu.sync_copy(src_hbm.at[pl.ds(base, _B2)], srcb.at[0])
        pltpu.sync_copy(dst_hbm.at[pl.ds(base, _B2)], dstb.at[0])

        def _mkidx(g, _g):
            v = srcb[0, pl.ds(g * 16, 16)]
            xib[0, pl.ds(g * 16, 16)] = v * 2 + c
            return 0
        lax.fori_loop(0, _B2 // 16, _mkidx, 0)

        cx = pltpu.async_copy(xl2_hbm.at[xib.at[0]], xbuf, sem0)
        cd = pltpu.async_copy(den_hbm.at[dstb.at[0]], dbuf, sem1)
        pltpu.sync_copy(s_hbm.at[pl.ds(base, _B2)], sbuf)
        cx.wait()
        cd.wait()

        def _edge(e, _e):
            srow = sbuf[e, :]
            drow = dbuf[e, :]
            arow = srow / (drow + 1e-16)
            for hh in range(4):
                a = jnp.sum(jnp.where(iota == 4 * c + hh, arow, 0.0))
                for q in range(2):
                    jj = hh * 2 + q
                    msg[e, pl.ds(jj * 16, 16)] = (
                        xbuf[e, pl.ds(jj * 16, 16)] * a)
            return 0
        lax.fori_loop(0, _B2, _edge, 0)

        pltpu.sync_copy(msg, acc_sh.at[dstb.at[0]], add=True)
        return 0

    lax.fori_loop(0, _CH2, _chunk, 0)
    plsc.subcore_barrier()
    for t in range(5):
        r0 = sidx * _RPS + t * (_RPS // 5)
        pltpu.sync_copy(acc_sh.at[pl.ds(r0, _RPS // 5)], stg)
        pltpu.sync_copy(stg, out_hbm.at[c, pl.ds(r0, _RPS // 5)])


@functools.partial(
    pl.kernel,
    out_type=jax.ShapeDtypeStruct((_NCORE, _NP, 128), jnp.float32),
    mesh=plsc.VectorSubcoreMesh(core_axis_name="c", subcore_axis_name="s"),
    compiler_params=pltpu.CompilerParams(use_tc_tiling_on_sc=False, needs_layout_passes=False),
    scratch_types=[
        pltpu.VMEM((1, _B2), jnp.int32),
        pltpu.VMEM((1, _B2), jnp.int32),
        pltpu.VMEM((1, _B2), jnp.int32),
        pltpu.VMEM((_B2, 128), jnp.float32),
        pltpu.VMEM((_B2, 128), jnp.float32),
        pltpu.VMEM((_B2, 16), jnp.float32),
        pltpu.VMEM((_B2, 16), jnp.float32),
        pltpu.VMEM((_RPS // 5, 128), jnp.float32),
        pltpu.VMEM_SHARED((_NP, 128), jnp.float32),
        pltpu.SemaphoreType.DMA,
        pltpu.SemaphoreType.DMA,
    ],
)
def _pass2(xl2_hbm, s_hbm, den_hbm, src_hbm, dst_hbm, out_hbm,
           srcb, dstb, xib, xbuf, msg, dbuf, sbuf, stg,
           acc_sh, sem0, sem1):
    _pass2_body(xl2_hbm, s_hbm, den_hbm, src_hbm, dst_hbm, out_hbm,
                srcb, dstb, xib, xbuf, msg, dbuf, sbuf, stg,
                acc_sh, sem0, sem1)


def _elu(x):
    return jnp.where(x > 0, x, jnp.exp(x) - 1.0)


def _mlp_body(o0_ref, o1_ref, ba_ref, bb_ref, w1a_ref, w1b_ref, b1_ref,
              w2_ref, b2_ref, y_ref):
    h0 = _elu(o0_ref[...] + ba_ref[...])
    h1 = _elu(o1_ref[...] + bb_ref[...])
    t = (jnp.dot(h0, w1a_ref[...], preferred_element_type=jnp.float32)
         + jnp.dot(h1, w1b_ref[...], preferred_element_type=jnp.float32)
         + b1_ref[...])
    t = _elu(t)
    y_ref[...] = jnp.dot(t, w2_ref[...],
                         preferred_element_type=jnp.float32) + b2_ref[...]


def _mlp(o0, o1, ba, bb, w1a, w1b, b12, W2, b22):
    blk = 400
    return pl.pallas_call(
        _mlp_body,
        grid=(_N // blk,),
        in_specs=[
            pl.BlockSpec((blk, 128), lambda i: (i, 0)),
            pl.BlockSpec((blk, 128), lambda i: (i, 0)),
            pl.BlockSpec((1, 128), lambda i: (0, 0)),
            pl.BlockSpec((1, 128), lambda i: (0, 0)),
            pl.BlockSpec((128, _C), lambda i: (0, 0)),
            pl.BlockSpec((128, _C), lambda i: (0, 0)),
            pl.BlockSpec((1, _C), lambda i: (0, 0)),
            pl.BlockSpec((_C, 128), lambda i: (0, 0)),
            pl.BlockSpec((1, 128), lambda i: (0, 0)),
        ],
        out_specs=pl.BlockSpec((blk, 128), lambda i: (i, 0)),
        out_shape=jax.ShapeDtypeStruct((_N, 128), jnp.float32),
    )(o0, o1, ba, bb, w1a, w1b, b12, W2, b22)


def kernel(x, edge_index, W_l, b_l, W_r, b_r, att, bias, W1, b1, W2, b2):
    src = edge_index[0]
    dst = edge_index[1]
    xl, xr = _proj(x, W_l, W_r, b_l.reshape(1, -1), b_r.reshape(1, -1))
    s, dp = _pass1(xl, xr, src, dst, att.reshape(-1))
    den = _combine(dp.reshape(2, _NP * 16 // 128, 128)).reshape(_NP, 16)
    o = _pass2(xl.reshape(2 * _N, 128), s, den, src, dst)
    y = _mlp(o[0, :_N], o[1, :_N],
             bias[:128].reshape(1, 128), bias[128:].reshape(1, 128),
             W1[:128], W1[128:], b1.reshape(1, -1), W2, b2.reshape(1, -1))
    return y


# speedup vs baseline: 17.1762x; 1.0549x over previous
import functools

import jax
import jax.numpy as jnp
from jax import lax
from jax.experimental import pallas as pl
from jax.experimental.pallas import tpu as pltpu
from jax.experimental.pallas import tpu_sc as plsc

_N = 10000
_E = 320000
_H = 8
_C = 32
_HC = _H * _C
_NSUB = 16
_NCORE = 2
_B1 = 80
_CH1 = (_E // 32) // _B1
_B2 = 80
_CH2 = (_E // _NSUB) // _B2
_NP = 10240
_RPS = _NP // _NSUB


def _proj_body(x_ref, wl_ref, wr_ref, bl_ref, br_ref, xl_ref, xr_ref):
    x = x_ref[...]
    xl_ref[...] = jnp.dot(x, wl_ref[...],
                          preferred_element_type=jnp.float32) + bl_ref[...]
    xr_ref[...] = jnp.dot(x, wr_ref[...],
                          preferred_element_type=jnp.float32) + br_ref[...]


def _proj(x, W_l, W_r, bl2, br2):
    blk = 400
    grid = (_N // blk,)
    return pl.pallas_call(
        _proj_body,
        grid=grid,
        in_specs=[
            pl.BlockSpec((blk, 128), lambda i: (i, 0)),
            pl.BlockSpec((128, _HC), lambda i: (0, 0)),
            pl.BlockSpec((128, _HC), lambda i: (0, 0)),
            pl.BlockSpec((1, _HC), lambda i: (0, 0)),
            pl.BlockSpec((1, _HC), lambda i: (0, 0)),
        ],
        out_specs=[
            pl.BlockSpec((blk, _HC), lambda i: (i, 0)),
            pl.BlockSpec((blk, _HC), lambda i: (i, 0)),
        ],
        out_shape=[
            jax.ShapeDtypeStruct((_N, _HC), jnp.float32),
            jax.ShapeDtypeStruct((_N, _HC), jnp.float32),
        ],
    )(x, W_l, W_r, bl2, br2)


def _pass1_body(xl_hbm, xr_hbm, src_hbm, dst_hbm, att_hbm,
                s_hbm, dp_hbm,
                srcb, dstb, bufL, bufR, s16, attb, stg,
                den_sh, sem0, sem1):
    c = lax.axis_index("c")
    sidx = lax.axis_index("s")
    wid = sidx * _NCORE + c
    i32 = jnp.int32
    zero16 = jnp.zeros((16,), jnp.float32)
    iota = lax.iota(i32, 16)

    pltpu.sync_copy(att_hbm, attb)
    attv = [attb[pl.ds(jj * 16, 16)] for jj in range(_HC // 16)]

    def _z_stg(i, _):
        stg[i, :] = zero16
        return 0
    lax.fori_loop(0, _RPS, _z_stg, 0)

    pltpu.sync_copy(stg, den_sh.at[pl.ds(sidx * _RPS, _RPS)])
    plsc.subcore_barrier()

    lane15 = jnp.broadcast_to(15, (16,)).astype(i32)

    def _chunk(k, _):
        base = wid * (_E // 32) + k * _B1
        pltpu.sync_copy(src_hbm.at[pl.ds(base, _B1)], srcb.at[0])
        pltpu.sync_copy(dst_hbm.at[pl.ds(base, _B1)], dstb.at[0])
        cl = pltpu.async_copy(xl_hbm.at[srcb.at[0]], bufL, sem0)
        cr = pltpu.async_copy(xr_hbm.at[dstb.at[0]], bufR, sem1)
        cl.wait()
        cr.wait()

        def _edge(e, _e):
            svec = jnp.full((16,), -1e30, jnp.float32)
            for h in range(_H):
                j0 = h * _C
                a0 = bufL[e, pl.ds(j0, 16)] + bufR[e, pl.ds(j0, 16)]
                a1 = bufL[e, pl.ds(j0 + 16, 16)] + bufR[e, pl.ds(j0 + 16, 16)]
                a0 = jnp.maximum(a0, 0.2 * a0) * attv[2 * h]
                a1 = jnp.maximum(a1, 0.2 * a1) * attv[2 * h + 1]
                cs = plsc.cumsum(a0 + a1)
                s = cs.at[lane15].get(mode="promise_in_bounds")
                svec = jnp.where(iota == h, s, svec)
            s16[e, :] = jnp.exp(svec)
            return 0

        lax.fori_loop(0, _B1, _edge, 0)
        pltpu.sync_copy(s16, s_hbm.at[pl.ds(base, _B1)])
        pltpu.sync_copy(s16, den_sh.at[dstb.at[0]], add=True)
        return 0

    lax.fori_loop(0, _CH1, _chunk, 0)
    plsc.subcore_barrier()
    r0 = sidx * _RPS
    pltpu.sync_copy(den_sh.at[pl.ds(r0, _RPS)], stg)
    pltpu.sync_copy(stg, dp_hbm.at[c, pl.ds(r0, _RPS)])


@functools.partial(
    pl.kernel,
    out_type=[
        jax.ShapeDtypeStruct((_E, 16), jnp.float32),
        jax.ShapeDtypeStruct((_NCORE, _NP, 16), jnp.float32),
    ],
    mesh=plsc.VectorSubcoreMesh(core_axis_name="c", subcore_axis_name="s"),
    compiler_params=pltpu.CompilerParams(use_tc_tiling_on_sc=False, needs_layout_passes=False),
    scratch_types=[
        pltpu.VMEM((1, _B1), jnp.int32),
        pltpu.VMEM((1, _B1), jnp.int32),
        pltpu.VMEM((_B1, _HC), jnp.float32),
        pltpu.VMEM((_B1, _HC), jnp.float32),
        pltpu.VMEM((_B1, 16), jnp.float32),
        pltpu.VMEM((_HC,), jnp.float32),
        pltpu.VMEM((_RPS, 16), jnp.float32),
        pltpu.VMEM_SHARED((_NP, 16), jnp.float32),
        pltpu.SemaphoreType.DMA,
        pltpu.SemaphoreType.DMA,
    ],
)
def _pass1(xl_hbm, xr_hbm, src_hbm, dst_hbm, att_hbm, s_hbm, dp_hbm,
           srcb, dstb, bufL, bufR, s16, attb, stg,
           den_sh, sem0, sem1):
    _pass1_body(xl_hbm, xr_hbm, src_hbm, dst_hbm, att_hbm, s_hbm, dp_hbm,
                srcb, dstb, bufL, bufR, s16, attb, stg,
                den_sh, sem0, sem1)


def _comb_body(dp_ref, out_ref):
    out_ref[...] = dp_ref[0] + dp_ref[1]


def _combine(dp3):
    return pl.pallas_call(
        _comb_body,
        out_shape=jax.ShapeDtypeStruct((_NP * 16 // 128, 128), jnp.float32),
    )(dp3)


def _pass2_body(xl2_hbm, s_hbm, den_hbm, src_hbm, dst_hbm, out_hbm,
                srcb, dstb, xib, xbuf, msg, dbuf, sbuf, stg,
                acc_sh, sem0, sem1):
    c = lax.axis_index("c")
    sidx = lax.axis_index("s")
    i32 = jnp.int32
    zero16 = jnp.zeros((16,), jnp.float32)
    iota = lax.iota(i32, 16)

    def _z_stg(i, _):
        for u in range(8):
            stg[i, pl.ds(u * 16, 16)] = zero16
        return 0
    lax.fori_loop(0, _RPS // 5, _z_stg, 0)
    for t in range(5):
        pltpu.sync_copy(
            stg, acc_sh.at[pl.ds(sidx * _RPS + t * (_RPS // 5), _RPS // 5)])
    plsc.subcore_barrier()

    def _chunk(k, _):
        base = sidx * (_E // _NSUB) + k * _B2
        pltpu.sync_copy(src_hbm.at[pl.ds(base, _B2)], srcb.at[0])
        pltpu.sync_copy(dst_hbm.at[pl.ds(base, _B2)], dstb.at[0])

        def _mkidx(g, _g):
            v = srcb[0, pl.ds(g * 16, 16)]
            xib[0, pl.ds(g * 16, 16)] = v * 2 + c
            return 0
        lax.fori_loop(0, _B2 // 16, _mkidx, 0)

        cx = pltpu.async_copy(xl2_hbm.at[xib.at[0]], xbuf, sem0)
        cd = pltpu.async_copy(den_hbm.at[dstb.at[0]], dbuf, sem1)
        pltpu.sync_copy(s_hbm.at[pl.ds(base, _B2)], sbuf)
        cx.wait()
        cd.wait()

        def _edge(e, _e):
            srow = sbuf[e, :]
            drow = dbuf[e, :]
            arow = srow / (drow + 1e-16)
            for hh in range(4):
                hv = jnp.broadcast_to(4 * c + hh, (16,)).astype(i32)
                a = arow.at[hv].get(mode="promise_in_bounds")
                for q in range(2):
                    jj = hh * 2 + q
                    msg[e, pl.ds(jj * 16, 16)] = (
                        xbuf[e, pl.ds(jj * 16, 16)] * a)
            return 0
        lax.fori_loop(0, _B2, _edge, 0)

        pltpu.sync_copy(msg, acc_sh.at[dstb.at[0]], add=True)
        return 0

    lax.fori_loop(0, _CH2, _chunk, 0)
    plsc.subcore_barrier()
    for t in range(5):
        r0 = sidx * _RPS + t * (_RPS // 5)
        pltpu.sync_copy(acc_sh.at[pl.ds(r0, _RPS // 5)], stg)
        pltpu.sync_copy(stg, out_hbm.at[c, pl.ds(r0, _RPS // 5)])


@functools.partial(
    pl.kernel,
    out_type=jax.ShapeDtypeStruct((_NCORE, _NP, 128), jnp.float32),
    mesh=plsc.VectorSubcoreMesh(core_axis_name="c", subcore_axis_name="s"),
    compiler_params=pltpu.CompilerParams(use_tc_tiling_on_sc=False, needs_layout_passes=False),
    scratch_types=[
        pltpu.VMEM((1, _B2), jnp.int32),
        pltpu.VMEM((1, _B2), jnp.int32),
        pltpu.VMEM((1, _B2), jnp.int32),
        pltpu.VMEM((_B2, 128), jnp.float32),
        pltpu.VMEM((_B2, 128), jnp.float32),
        pltpu.VMEM((_B2, 16), jnp.float32),
        pltpu.VMEM((_B2, 16), jnp.float32),
        pltpu.VMEM((_RPS // 5, 128), jnp.float32),
        pltpu.VMEM_SHARED((_NP, 128), jnp.float32),
        pltpu.SemaphoreType.DMA,
        pltpu.SemaphoreType.DMA,
    ],
)
def _pass2(xl2_hbm, s_hbm, den_hbm, src_hbm, dst_hbm, out_hbm,
           srcb, dstb, xib, xbuf, msg, dbuf, sbuf, stg,
           acc_sh, sem0, sem1):
    _pass2_body(xl2_hbm, s_hbm, den_hbm, src_hbm, dst_hbm, out_hbm,
                srcb, dstb, xib, xbuf, msg, dbuf, sbuf, stg,
                acc_sh, sem0, sem1)


def _elu(x):
    return jnp.where(x > 0, x, jnp.exp(x) - 1.0)


def _mlp_body(o0_ref, o1_ref, ba_ref, bb_ref, w1a_ref, w1b_ref, b1_ref,
              w2_ref, b2_ref, y_ref):
    h0 = _elu(o0_ref[...] + ba_ref[...])
    h1 = _elu(o1_ref[...] + bb_ref[...])
    t = (jnp.dot(h0, w1a_ref[...], preferred_element_type=jnp.float32)
         + jnp.dot(h1, w1b_ref[...], preferred_element_type=jnp.float32)
         + b1_ref[...])
    t = _elu(t)
    y_ref[...] = jnp.dot(t, w2_ref[...],
                         preferred_element_type=jnp.float32) + b2_ref[...]


def _mlp(o0, o1, ba, bb, w1a, w1b, b12, W2, b22):
    blk = 400
    return pl.pallas_call(
        _mlp_body,
        grid=(_N // blk,),
        in_specs=[
            pl.BlockSpec((blk, 128), lambda i: (i, 0)),
            pl.BlockSpec((blk, 128), lambda i: (i, 0)),
            pl.BlockSpec((1, 128), lambda i: (0, 0)),
            pl.BlockSpec((1, 128), lambda i: (0, 0)),
            pl.BlockSpec((128, _C), lambda i: (0, 0)),
            pl.BlockSpec((128, _C), lambda i: (0, 0)),
            pl.BlockSpec((1, _C), lambda i: (0, 0)),
            pl.BlockSpec((_C, 128), lambda i: (0, 0)),
            pl.BlockSpec((1, 128), lambda i: (0, 0)),
        ],
        out_specs=pl.BlockSpec((blk, 128), lambda i: (i, 0)),
        out_shape=jax.ShapeDtypeStruct((_N, 128), jnp.float32),
    )(o0, o1, ba, bb, w1a, w1b, b12, W2, b22)


def kernel(x, edge_index, W_l, b_l, W_r, b_r, att, bias, W1, b1, W2, b2):
    src = edge_index[0]
    dst = edge_index[1]
    xl, xr = _proj(x, W_l, W_r, b_l.reshape(1, -1), b_r.reshape(1, -1))
    s, dp = _pass1(xl, xr, src, dst, att.reshape(-1))
    den = _combine(dp.reshape(2, _NP * 16 // 128, 128)).reshape(_NP, 16)
    o = _pass2(xl.reshape(2 * _N, 128), s, den, src, dst)
    y = _mlp(o[0, :_N], o[1, :_N],
             bias[:128].reshape(1, 128), bias[128:].reshape(1, 128),
             W1[:128], W1[128:], b1.reshape(1, -1), W2, b2.reshape(1, -1))
    return y
